# baseline (speedup 1.0000x reference)
import jax
import jax.numpy as jnp
from jax import lax
from jax.experimental import pallas as pl
from jax.experimental.pallas import tpu as pltpu
from jax.experimental.pallas import tpu_sc as plsc

N = 10000
E = 160000
IN = 256
HID = 512
H = 8
OC = 64
OUTD = 256
NEG = 0.2

NPAD = 10112
GRID = NPAD // 128
NTILES = 32
EB = 32
NB = 168
ETOT = E + N
EPAD = NTILES * NB * EB
STRIPE = NPAD // 16
DROWS = 80

f32 = jnp.float32
i32 = jnp.int32


def _tc_head(k_dim, m_dim, nchunk):
    def body(x_ref, w_ref, ams_ref, amd_ref, *out_refs):
        h = jnp.dot(x_ref[...], w_ref[...], preferred_element_type=f32)
        for c in range(nchunk):
            out_refs[c][...] = h[:, c * 64:(c + 1) * 64]
        dn = (((1,), (1,)), ((), ()))
        out_refs[nchunk][...] = lax.dot_general(
            ams_ref[...], h, dn, preferred_element_type=f32)
        out_refs[nchunk + 1][...] = lax.dot_general(
            amd_ref[...], h, dn, preferred_element_type=f32)

    out_shape = ([jax.ShapeDtypeStruct((NPAD, 64), f32) for _ in range(nchunk)]
                 + [jax.ShapeDtypeStruct((8, NPAD), f32)] * 2)
    in_specs = [
        pl.BlockSpec((128, k_dim), lambda i: (i, 0)),
        pl.BlockSpec((k_dim, m_dim), lambda i: (0, 0)),
        pl.BlockSpec((8, m_dim), lambda i: (0, 0)),
        pl.BlockSpec((8, m_dim), lambda i: (0, 0)),
    ]
    out_specs = ([pl.BlockSpec((128, 64), lambda i: (i, 0))] * nchunk
                 + [pl.BlockSpec((8, 128), lambda i: (0, i))] * 2)
    return pl.pallas_call(body, grid=(GRID,), in_specs=in_specs,
                          out_specs=out_specs, out_shape=out_shape)


def _tc_fin_next(nchunk_prev, m_prev, k_dim, m_next, nchunk_next,
                 emit_x, relu):
    def body(acc_ref, den_ref, b_ref, w_ref, ams_ref, amd_ref, *out_refs):
        den = jnp.maximum(den_ref[0] + den_ref[1], 1e-16)
        parts = []
        for c in range(nchunk_prev):
            acc_c = acc_ref[0, c] + acc_ref[1, c]
            parts.append(acc_c / den[:, c:c + 1])
        xn = jnp.concatenate(parts, axis=1) + b_ref[...]
        if relu:
            xn = jnp.maximum(xn, 0.0)
        off = 0
        if emit_x:
            out_refs[0][...] = xn
            off = 1
        h = jnp.dot(xn, w_ref[...], preferred_element_type=f32)
        for c in range(nchunk_next):
            out_refs[off + c][...] = h[:, c * 64:(c + 1) * 64]
        dn = (((1,), (1,)), ((), ()))
        out_refs[off + nchunk_next][...] = lax.dot_general(
            ams_ref[...], h, dn, preferred_element_type=f32)
        out_refs[off + nchunk_next + 1][...] = lax.dot_general(
            amd_ref[...], h, dn, preferred_element_type=f32)

    out_shape = []
    out_specs = []
    if emit_x:
        out_shape.append(jax.ShapeDtypeStruct((NPAD, m_prev), f32))
        out_specs.append(pl.BlockSpec((128, m_prev), lambda i: (i, 0)))
    out_shape += [jax.ShapeDtypeStruct((NPAD, 64), f32)
                  for _ in range(nchunk_next)]
    out_specs += [pl.BlockSpec((128, 64), lambda i: (i, 0))] * nchunk_next
    out_shape += [jax.ShapeDtypeStruct((8, NPAD), f32)] * 2
    out_specs += [pl.BlockSpec((8, 128), lambda i: (0, i))] * 2
    in_specs = [
        pl.BlockSpec((2, nchunk_prev, 128, 64), lambda i: (0, 0, i, 0)),
        pl.BlockSpec((2, 128, 128), lambda i: (0, i, 0)),
        pl.BlockSpec((1, m_prev), lambda i: (0, 0)),
        pl.BlockSpec((k_dim, m_next), lambda i: (0, 0)),
        pl.BlockSpec((8, m_next), lambda i: (0, 0)),
        pl.BlockSpec((8, m_next), lambda i: (0, 0)),
    ]
    return pl.pallas_call(body, grid=(GRID,), in_specs=in_specs,
                          out_specs=out_specs, out_shape=out_shape)


def _tc_fin_last():
    def body(acc_ref, den_ref, b_ref, out_ref):
        den = jnp.maximum(den_ref[0] + den_ref[1], 1e-16)[:, 0:1]
        o = jnp.concatenate(
            [(acc_ref[0, c] + acc_ref[1, c]) / den for c in range(4)], axis=1)
        out_ref[...] = o + b_ref[...]

    in_specs = [
        pl.BlockSpec((2, 4, 128, 64), lambda i: (0, 0, i, 0)),
        pl.BlockSpec((2, 128, 128), lambda i: (0, i, 0)),
        pl.BlockSpec((1, OUTD), lambda i: (0, 0)),
    ]
    return pl.pallas_call(
        body, grid=(GRID,), in_specs=in_specs,
        out_specs=pl.BlockSpec((128, OUTD), lambda i: (i, 0)),
        out_shape=jax.ShapeDtypeStruct((NPAD, OUTD), f32))


def _sc_agg(passes):
    npass = len(passes)
    mesh = plsc.VectorSubcoreMesh(core_axis_name="c", subcore_axis_name="s")
    scratch = [
        pltpu.VMEM((NB, EB), i32),
        pltpu.VMEM((NB, EB), i32),
        pltpu.VMEM((NPAD,), f32),
        pltpu.VMEM((NPAD,), f32),
        pltpu.VMEM((EB, 64), f32),
        pltpu.VMEM((EB, 64), f32),
        pltpu.VMEM((EB, 64), f32),
        pltpu.VMEM((EB + 16,), f32),
        pltpu.SemaphoreType.DMA,
        pltpu.SemaphoreType.DMA,
        pltpu.SemaphoreType.DMA,
        pltpu.VMEM((DROWS, 128), f32),
        pltpu.VMEM((5, 128), f32),
        pltpu.VMEM((DROWS,), i32),
        pltpu.VMEM_SHARED((NPAD, 64), f32),
        pltpu.VMEM_SHARED((NPAD, 64), f32),
        pltpu.VMEM_SHARED((DROWS, 128), f32),
    ]
    out_type = [jax.ShapeDtypeStruct((2, npass, NPAD, 64), f32),
                jax.ShapeDtypeStruct((2, npass, DROWS, 128), f32)]

    def body(*refs):
        hc = refs[:npass]
        asT, adT, srcT, dstT, zeros_hbm = refs[npass:npass + 5]
        out_acc, out_den = refs[npass + 5:npass + 7]
        (src_v, dst_v, asA, adA, rb0, rb1, rb2, w_v, sm0, sm1, sm2,
         den_v, zden, idxm, hc_sh, acc_sh, den_sh) = refs[npass + 7:]
        bufs = (rb0, rb1, rb2)
        sems = (sm0, sm1, sm2)
        cid = lax.axis_index("c")
        sid = lax.axis_index("s")
        wid = cid * 16 + sid
        z16 = jnp.zeros((16,), f32)
        io16 = jnp.arange(16, dtype=i32)

        for g in range(5 * 8):
            zden[g >> 3, pl.ds((g & 7) * 16, 16)] = z16
        for g in range(5):
            idxm[pl.ds(g * 16, 16)] = io16 + g * 16
        pltpu.sync_copy(srcT.at[wid], src_v)
        pltpu.sync_copy(dstT.at[wid], dst_v)

        for p, (hidx, do_den) in enumerate(passes):
            pltpu.sync_copy(asT.at[hidx], asA)
            pltpu.sync_copy(adT.at[hidx], adA)
            pltpu.sync_copy(hc[p].at[pl.ds(sid * STRIPE, STRIPE)],
                            hc_sh.at[pl.ds(sid * STRIPE, STRIPE)])
            pltpu.sync_copy(zeros_hbm.at[pl.ds(sid * STRIPE, STRIPE)],
                            acc_sh.at[pl.ds(sid * STRIPE, STRIPE)])
            pltpu.sync_copy(zden, den_sh.at[pl.ds(sid * 5, 5)])
            if do_den:
                def zd(t, _):
                    den_v[t >> 3, pl.ds((t & 7) * 16, 16)] = z16
                    return 0
                lax.fori_loop(0, DROWS * 8, zd, 0)
            plsc.subcore_barrier()

            def do_batch(j, buf, sem):
                for g in range(EB // 16):
                    s16 = src_v[j, pl.ds(g * 16, 16)]
                    d16 = dst_v[j, pl.ds(g * 16, 16)]
                    e0 = (plsc.load_gather(asA, [s16])
                          + plsc.load_gather(adA, [d16]))
                    e0 = jnp.where(e0 > 0, e0, NEG * e0)
                    w0 = jnp.exp(e0)
                    w_v[pl.ds(g * 16, 16)] = w0
                    if do_den:
                        plsc.addupdate_scatter(
                            den_v, [d16 >> 7, d16 & 127], w0)
                pltpu.make_async_copy(hc_sh.at[src_v.at[j]], buf, sem).wait()

                @plsc.parallel_loop(0, EB, unroll=8)
                def _(b):
                    v0 = plsc.load_gather(w_v, [jnp.full((16,), b, i32)])
                    for k in range(4):
                        buf[b, pl.ds(k * 16, 16)] = (
                            buf[b, pl.ds(k * 16, 16)] * v0)
                pltpu.sync_copy(buf, acc_sh.at[dst_v.at[j]], add=True)

            for t in range(2):
                pltpu.async_copy(hc_sh.at[src_v.at[t]], bufs[t], sems[t])

            def tri(qq, _):
                j0 = 3 * qq
                for t in range(3):
                    jt = j0 + t

                    @pl.when(jt + 2 < NB)
                    def _():
                        pltpu.async_copy(hc_sh.at[src_v.at[jt + 2]],
                                         bufs[(t + 2) % 3], sems[(t + 2) % 3])
                    do_batch(jt, bufs[t], sems[t])
                return 0
            lax.fori_loop(0, NB // 3, tri, 0)
            plsc.subcore_barrier()
            if do_den:
                pltpu.sync_copy(den_v, den_sh.at[idxm], add=True)
            plsc.subcore_barrier()
            pltpu.sync_copy(acc_sh.at[pl.ds(sid * STRIPE, STRIPE)],
                            out_acc.at[cid, p, pl.ds(sid * STRIPE, STRIPE)])
            if do_den:
                @pl.when(sid == 0)
                def _():
                    pltpu.sync_copy(den_sh, out_den.at[cid, p])
            plsc.subcore_barrier()

    return pl.kernel(
        body, out_type=out_type, mesh=mesh, scratch_types=scratch,
        compiler_params=pltpu.CompilerParams(needs_layout_passes=False,
                                             use_tc_tiling_on_sc=False))


def _block_diag_a(a, heads, oc, m):
    A = jnp.zeros((8, m), f32)
    af = a.reshape(heads, oc)
    for hh in range(heads):
        A = A.at[hh, hh * oc:(hh + 1) * oc].set(af[hh])
    return A


def _den_tc(den_out, nheads):
    d = den_out.reshape(2, den_out.shape[1], DROWS * 128)[:, :nheads, :NPAD]
    d = d.transpose(0, 2, 1)
    return jnp.pad(d, ((0, 0), (0, 0), (0, 128 - nheads)))


_tc0 = _tc_head(IN, HID, 8)
_tc1 = _tc_fin_next(8, HID, HID, HID, 8, emit_x=False, relu=True)
_tc2 = _tc_fin_next(8, HID, HID, OUTD, 4, emit_x=True, relu=True)
_tc3 = _tc_fin_last()
_sc01 = _sc_agg([(hh, True) for hh in range(8)])
_sc2 = _sc_agg([(0, True)] + [(0, False)] * 3)


def kernel(x, edge_index, W0, as0, ad0, b0, W1, as1, ad1, b1,
           W2, as2, ad2, b2):
    loop = jnp.arange(N, dtype=edge_index.dtype)
    src = jnp.concatenate([edge_index[0], loop]).astype(i32)
    dst = jnp.concatenate([edge_index[1], loop]).astype(i32)
    srcT = jnp.pad(src, (0, EPAD - ETOT)).reshape(NTILES, NB, EB)
    dstT = jnp.pad(dst, (0, EPAD - ETOT),
                   constant_values=N).reshape(NTILES, NB, EB)
    xp = jnp.pad(x, ((0, NPAD - N), (0, 0)))

    amS0 = _block_diag_a(as0, H, OC, HID)
    amD0 = _block_diag_a(ad0, H, OC, HID)
    amS1 = _block_diag_a(as1, H, OC, HID)
    amD1 = _block_diag_a(ad1, H, OC, HID)
    amS2 = _block_diag_a(as2, 1, OUTD, OUTD)
    amD2 = _block_diag_a(ad2, 1, OUTD, OUTD)

    zhbm = jnp.zeros((NPAD, 64), f32)

    o = _tc0(xp, W0, amS0, amD0)
    acc, den = _sc01(*o[:8], o[8], o[9], srcT, dstT, zhbm)

    o = _tc1(acc, _den_tc(den, 8), b0.reshape(1, HID), W1, amS1, amD1)
    acc, den = _sc01(*o[:8], o[8], o[9], srcT, dstT, zhbm)

    o = _tc2(acc, _den_tc(den, 8), b1.reshape(1, HID), W2, amS2, amD2)
    xpen = o[0]
    acc, den = _sc2(*o[1:5], o[5], o[6], srcT, dstT, zhbm)

    out = _tc3(acc, _den_tc(den, 1), b2.reshape(1, OUTD))
    return (xpen[:N], out[:N])


# speedup vs baseline: 1.0035x; 1.0035x over previous
import jax
import jax.numpy as jnp
from jax import lax
from jax.experimental import pallas as pl
from jax.experimental.pallas import tpu as pltpu
from jax.experimental.pallas import tpu_sc as plsc

N = 10000
E = 160000
IN = 256
HID = 512
H = 8
OC = 64
OUTD = 256
NEG = 0.2

NPAD = 10112
GRID = NPAD // 128
NTILES = 32
EB = 32
NB = 168
ETOT = E + N
EPAD = NTILES * NB * EB
STRIPE = NPAD // 16
DROWS = 80

f32 = jnp.float32
i32 = jnp.int32


def _tc_head(k_dim, m_dim, nchunk):
    def body(x_ref, w_ref, ams_ref, amd_ref, *out_refs):
        h = jnp.dot(x_ref[...], w_ref[...], preferred_element_type=f32)
        for c in range(nchunk):
            out_refs[c][...] = h[:, c * 64:(c + 1) * 64]
        dn = (((1,), (1,)), ((), ()))
        out_refs[nchunk][...] = lax.dot_general(
            ams_ref[...], h, dn, preferred_element_type=f32)
        out_refs[nchunk + 1][...] = lax.dot_general(
            amd_ref[...], h, dn, preferred_element_type=f32)

    out_shape = ([jax.ShapeDtypeStruct((NPAD, 64), f32) for _ in range(nchunk)]
                 + [jax.ShapeDtypeStruct((8, NPAD), f32)] * 2)
    in_specs = [
        pl.BlockSpec((128, k_dim), lambda i: (i, 0)),
        pl.BlockSpec((k_dim, m_dim), lambda i: (0, 0)),
        pl.BlockSpec((8, m_dim), lambda i: (0, 0)),
        pl.BlockSpec((8, m_dim), lambda i: (0, 0)),
    ]
    out_specs = ([pl.BlockSpec((128, 64), lambda i: (i, 0))] * nchunk
                 + [pl.BlockSpec((8, 128), lambda i: (0, i))] * 2)
    return pl.pallas_call(body, grid=(GRID,), in_specs=in_specs,
                          out_specs=out_specs, out_shape=out_shape)


def _tc_fin_next(nchunk_prev, m_prev, k_dim, m_next, nchunk_next,
                 emit_x, relu):
    def body(acc_ref, den_ref, b_ref, w_ref, ams_ref, amd_ref, *out_refs):
        den = jnp.maximum(den_ref[0] + den_ref[1], 1e-16)
        parts = []
        for c in range(nchunk_prev):
            acc_c = acc_ref[0, c] + acc_ref[1, c]
            parts.append(acc_c / den[:, c:c + 1])
        xn = jnp.concatenate(parts, axis=1) + b_ref[...]
        if relu:
            xn = jnp.maximum(xn, 0.0)
        off = 0
        if emit_x:
            out_refs[0][...] = xn
            off = 1
        h = jnp.dot(xn, w_ref[...], preferred_element_type=f32)
        for c in range(nchunk_next):
            out_refs[off + c][...] = h[:, c * 64:(c + 1) * 64]
        dn = (((1,), (1,)), ((), ()))
        out_refs[off + nchunk_next][...] = lax.dot_general(
            ams_ref[...], h, dn, preferred_element_type=f32)
        out_refs[off + nchunk_next + 1][...] = lax.dot_general(
            amd_ref[...], h, dn, preferred_element_type=f32)

    out_shape = []
    out_specs = []
    if emit_x:
        out_shape.append(jax.ShapeDtypeStruct((NPAD, m_prev), f32))
        out_specs.append(pl.BlockSpec((128, m_prev), lambda i: (i, 0)))
    out_shape += [jax.ShapeDtypeStruct((NPAD, 64), f32)
                  for _ in range(nchunk_next)]
    out_specs += [pl.BlockSpec((128, 64), lambda i: (i, 0))] * nchunk_next
    out_shape += [jax.ShapeDtypeStruct((8, NPAD), f32)] * 2
    out_specs += [pl.BlockSpec((8, 128), lambda i: (0, i))] * 2
    in_specs = [
        pl.BlockSpec((2, nchunk_prev, 128, 64), lambda i: (0, 0, i, 0)),
        pl.BlockSpec((2, 128, 128), lambda i: (0, i, 0)),
        pl.BlockSpec((1, m_prev), lambda i: (0, 0)),
        pl.BlockSpec((k_dim, m_next), lambda i: (0, 0)),
        pl.BlockSpec((8, m_next), lambda i: (0, 0)),
        pl.BlockSpec((8, m_next), lambda i: (0, 0)),
    ]
    return pl.pallas_call(body, grid=(GRID,), in_specs=in_specs,
                          out_specs=out_specs, out_shape=out_shape)


def _tc_fin_last():
    def body(acc_ref, den_ref, b_ref, out_ref):
        den = jnp.maximum(den_ref[0] + den_ref[1], 1e-16)[:, 0:1]
        o = jnp.concatenate(
            [(acc_ref[0, c] + acc_ref[1, c]) / den for c in range(4)], axis=1)
        out_ref[...] = o + b_ref[...]

    in_specs = [
        pl.BlockSpec((2, 4, 128, 64), lambda i: (0, 0, i, 0)),
        pl.BlockSpec((2, 128, 128), lambda i: (0, i, 0)),
        pl.BlockSpec((1, OUTD), lambda i: (0, 0)),
    ]
    return pl.pallas_call(
        body, grid=(GRID,), in_specs=in_specs,
        out_specs=pl.BlockSpec((128, OUTD), lambda i: (i, 0)),
        out_shape=jax.ShapeDtypeStruct((NPAD, OUTD), f32))


def _sc_agg(passes):
    npass = len(passes)
    mesh = plsc.VectorSubcoreMesh(core_axis_name="c", subcore_axis_name="s")
    scratch = [
        pltpu.VMEM((NB, EB), i32),
        pltpu.VMEM((NB, EB), i32),
        pltpu.VMEM((NPAD,), f32),
        pltpu.VMEM((NPAD,), f32),
        pltpu.VMEM((EB, 64), f32),
        pltpu.VMEM((EB, 64), f32),
        pltpu.VMEM((EB, 64), f32),
        pltpu.VMEM((EB + 16,), f32),
        pltpu.SemaphoreType.DMA,
        pltpu.SemaphoreType.DMA,
        pltpu.SemaphoreType.DMA,
        pltpu.SemaphoreType.DMA,
        pltpu.SemaphoreType.DMA,
        pltpu.SemaphoreType.DMA,
        pltpu.VMEM((DROWS, 128), f32),
        pltpu.VMEM((5, 128), f32),
        pltpu.VMEM((DROWS,), i32),
        pltpu.VMEM_SHARED((NPAD, 64), f32),
        pltpu.VMEM_SHARED((NPAD, 64), f32),
        pltpu.VMEM_SHARED((DROWS, 128), f32),
    ]
    out_type = [jax.ShapeDtypeStruct((2, npass, NPAD, 64), f32),
                jax.ShapeDtypeStruct((2, npass, DROWS, 128), f32)]

    def body(*refs):
        hc = refs[:npass]
        asT, adT, srcT, dstT, zeros_hbm = refs[npass:npass + 5]
        out_acc, out_den = refs[npass + 5:npass + 7]
        (src_v, dst_v, asA, adA, rb0, rb1, rb2, w_v, sm0, sm1, sm2,
         ss0, ss1, ss2, den_v, zden, idxm, hc_sh, acc_sh,
         den_sh) = refs[npass + 7:]
        bufs = (rb0, rb1, rb2)
        sems = (sm0, sm1, sm2)
        ssems = (ss0, ss1, ss2)
        cid = lax.axis_index("c")
        sid = lax.axis_index("s")
        wid = cid * 16 + sid
        z16 = jnp.zeros((16,), f32)
        io16 = jnp.arange(16, dtype=i32)

        for g in range(5 * 8):
            zden[g >> 3, pl.ds((g & 7) * 16, 16)] = z16
        for g in range(5):
            idxm[pl.ds(g * 16, 16)] = io16 + g * 16
        pltpu.sync_copy(srcT.at[wid], src_v)
        pltpu.sync_copy(dstT.at[wid], dst_v)

        for p, (hidx, do_den) in enumerate(passes):
            pltpu.sync_copy(asT.at[hidx], asA)
            pltpu.sync_copy(adT.at[hidx], adA)
            pltpu.sync_copy(hc[p].at[pl.ds(sid * STRIPE, STRIPE)],
                            hc_sh.at[pl.ds(sid * STRIPE, STRIPE)])
            pltpu.sync_copy(zeros_hbm.at[pl.ds(sid * STRIPE, STRIPE)],
                            acc_sh.at[pl.ds(sid * STRIPE, STRIPE)])
            pltpu.sync_copy(zden, den_sh.at[pl.ds(sid * 5, 5)])
            if do_den:
                def zd(t, _):
                    den_v[t >> 3, pl.ds((t & 7) * 16, 16)] = z16
                    return 0
                lax.fori_loop(0, DROWS * 8, zd, 0)
            plsc.subcore_barrier()

            def do_batch(j, buf, sem, ssem):
                for g in range(EB // 16):
                    s16 = src_v[j, pl.ds(g * 16, 16)]
                    d16 = dst_v[j, pl.ds(g * 16, 16)]
                    e0 = (plsc.load_gather(asA, [s16])
                          + plsc.load_gather(adA, [d16]))
                    e0 = jnp.where(e0 > 0, e0, NEG * e0)
                    w0 = jnp.exp(e0)
                    w_v[pl.ds(g * 16, 16)] = w0
                    if do_den:
                        plsc.addupdate_scatter(
                            den_v, [d16 >> 7, d16 & 127], w0)
                pltpu.make_async_copy(hc_sh.at[src_v.at[j]], buf, sem).wait()

                @plsc.parallel_loop(0, EB, unroll=8)
                def _(b):
                    v0 = plsc.load_gather(w_v, [jnp.full((16,), b, i32)])
                    for k in range(4):
                        buf[b, pl.ds(k * 16, 16)] = (
                            buf[b, pl.ds(k * 16, 16)] * v0)
                pltpu.async_copy(buf, acc_sh.at[dst_v.at[j]], ssem,
                                 add=True)

            for t in range(2):
                pltpu.async_copy(hc_sh.at[src_v.at[t]], bufs[t], sems[t])

            def tri(qq, _):
                j0 = 3 * qq
                for t in range(3):
                    jt = j0 + t
                    b2 = (t + 2) % 3

                    @pl.when(jt + 2 < NB)
                    def _():
                        @pl.when(jt >= 1)
                        def _():
                            pltpu.make_async_copy(
                                bufs[b2], acc_sh.at[dst_v.at[jt]],
                                ssems[b2]).wait()
                        pltpu.async_copy(hc_sh.at[src_v.at[jt + 2]],
                                         bufs[b2], sems[b2])
                    do_batch(jt, bufs[t], sems[t], ssems[t])
                return 0
            lax.fori_loop(0, NB // 3, tri, 0)
            for t in range(3):
                pltpu.make_async_copy(bufs[t], acc_sh.at[dst_v.at[0]],
                                      ssems[t]).wait()
            plsc.subcore_barrier()
            if do_den:
                pltpu.sync_copy(den_v, den_sh.at[idxm], add=True)
            plsc.subcore_barrier()
            pltpu.sync_copy(acc_sh.at[pl.ds(sid * STRIPE, STRIPE)],
                            out_acc.at[cid, p, pl.ds(sid * STRIPE, STRIPE)])
            if do_den:
                @pl.when(sid == 0)
                def _():
                    pltpu.sync_copy(den_sh, out_den.at[cid, p])
            plsc.subcore_barrier()

    return pl.kernel(
        body, out_type=out_type, mesh=mesh, scratch_types=scratch,
        compiler_params=pltpu.CompilerParams(needs_layout_passes=False,
                                             use_tc_tiling_on_sc=False))


def _block_diag_a(a, heads, oc, m):
    A = jnp.zeros((8, m), f32)
    af = a.reshape(heads, oc)
    for hh in range(heads):
        A = A.at[hh, hh * oc:(hh + 1) * oc].set(af[hh])
    return A


def _den_tc(den_out, nheads):
    d = den_out.reshape(2, den_out.shape[1], DROWS * 128)[:, :nheads, :NPAD]
    d = d.transpose(0, 2, 1)
    return jnp.pad(d, ((0, 0), (0, 0), (0, 128 - nheads)))


_tc0 = _tc_head(IN, HID, 8)
_tc1 = _tc_fin_next(8, HID, HID, HID, 8, emit_x=False, relu=True)
_tc2 = _tc_fin_next(8, HID, HID, OUTD, 4, emit_x=True, relu=True)
_tc3 = _tc_fin_last()
_sc01 = _sc_agg([(hh, True) for hh in range(8)])
_sc2 = _sc_agg([(0, True)] + [(0, False)] * 3)


def kernel(x, edge_index, W0, as0, ad0, b0, W1, as1, ad1, b1,
           W2, as2, ad2, b2):
    loop = jnp.arange(N, dtype=edge_index.dtype)
    src = jnp.concatenate([edge_index[0], loop]).astype(i32)
    dst = jnp.concatenate([edge_index[1], loop]).astype(i32)
    srcT = jnp.pad(src, (0, EPAD - ETOT)).reshape(NTILES, NB, EB)
    dstT = jnp.pad(dst, (0, EPAD - ETOT),
                   constant_values=N).reshape(NTILES, NB, EB)
    xp = jnp.pad(x, ((0, NPAD - N), (0, 0)))

    amS0 = _block_diag_a(as0, H, OC, HID)
    amD0 = _block_diag_a(ad0, H, OC, HID)
    amS1 = _block_diag_a(as1, H, OC, HID)
    amD1 = _block_diag_a(ad1, H, OC, HID)
    amS2 = _block_diag_a(as2, 1, OUTD, OUTD)
    amD2 = _block_diag_a(ad2, 1, OUTD, OUTD)

    zhbm = jnp.zeros((NPAD, 64), f32)

    o = _tc0(xp, W0, amS0, amD0)
    acc, den = _sc01(*o[:8], o[8], o[9], srcT, dstT, zhbm)

    o = _tc1(acc, _den_tc(den, 8), b0.reshape(1, HID), W1, amS1, amD1)
    acc, den = _sc01(*o[:8], o[8], o[9], srcT, dstT, zhbm)

    o = _tc2(acc, _den_tc(den, 8), b1.reshape(1, HID), W2, amS2, amD2)
    xpen = o[0]
    acc, den = _sc2(*o[1:5], o[5], o[6], srcT, dstT, zhbm)

    out = _tc3(acc, _den_tc(den, 1), b2.reshape(1, OUTD))
    return (xpen[:N], out[:N])
